# baseline (speedup 1.0000x reference)
import functools

import jax
import jax.numpy as jnp
from jax import lax
from jax.experimental import pallas as pl
from jax.experimental.pallas import tpu as pltpu
from jax.experimental.pallas import tpu_sc as plsc

_VOCAB = 1000000
_EMBED = 64
_HIDDEN = 100
_OUT_PAD = 16
_BATCH = 4096
_SEQ = 200

_TC_BLK = 8000

_NC = 2
_NS = 16
_NW = _NC * _NS
_B_PER_W = _BATCH // _NW
_CHUNK = 100
_CHUNKS_PER_BATCH = _SEQ // _CHUNK


def _mlp_body(tab_ref, w1_ref, b1_ref, w2_ref, b2_ref, w3_ref, b3_ref, g_ref):
    h = jnp.tanh(
        jnp.dot(tab_ref[...], w1_ref[...], preferred_element_type=jnp.float32)
        + b1_ref[...]
    )
    h = jnp.tanh(
        jnp.dot(h, w2_ref[...], preferred_element_type=jnp.float32) + b2_ref[...]
    )
    g_ref[...] = (
        jnp.dot(h, w3_ref[...], preferred_element_type=jnp.float32) + b3_ref[...]
    )


def _compute_g(table, W1, b1, W2, b2, W3p, b3p):
    nblk = _VOCAB // _TC_BLK
    full = lambda shape: pl.BlockSpec(shape, lambda i: (0, 0))
    return pl.pallas_call(
        _mlp_body,
        grid=(nblk,),
        in_specs=[
            pl.BlockSpec((_TC_BLK, _EMBED), lambda i: (i, 0)),
            full((_EMBED, _HIDDEN)),
            full((1, _HIDDEN)),
            full((_HIDDEN, _HIDDEN)),
            full((1, _HIDDEN)),
            full((_HIDDEN, _OUT_PAD)),
            full((1, _OUT_PAD)),
        ],
        out_specs=pl.BlockSpec((_TC_BLK, _OUT_PAD), lambda i: (i, 0)),
        out_shape=jax.ShapeDtypeStruct((_VOCAB, _OUT_PAD), jnp.float32),
    )(table, W1, b1, W2, b2, W3p, b3p)


def _sc_body(g_hbm, idx_hbm, out_hbm, idx_v, rows_v, acc_v, sem):
    wid = lax.axis_index("s") * _NC + lax.axis_index("c")
    nrows = _B_PER_W * _CHUNKS_PER_BATCH
    pltpu.sync_copy(idx_hbm.at[pl.ds(wid * nrows, nrows)], idx_v)

    zero = jnp.zeros((_OUT_PAD,), jnp.float32)

    def sum_rows(acc):
        def inner(j, carry):
            a0, a1, a2, a3 = carry
            base = j * 4
            return (
                a0 + rows_v[base],
                a1 + rows_v[base + 1],
                a2 + rows_v[base + 2],
                a3 + rows_v[base + 3],
            )

        a0, a1, a2, a3 = lax.fori_loop(0, _CHUNK // 4, inner, (acc, zero, zero, zero))
        return a0 + a1 + a2 + a3

    def batch_body(b, carry):
        acc = zero
        for half in range(_CHUNKS_PER_BATCH):
            row = b * _CHUNKS_PER_BATCH + half
            pltpu.async_copy(g_hbm.at[idx_v.at[row]], rows_v, sem).wait()
            acc = sum_rows(acc)
        acc_v[b] = acc
        return carry

    lax.fori_loop(0, _B_PER_W, batch_body, 0)
    pltpu.sync_copy(acc_v, out_hbm.at[pl.ds(wid * _B_PER_W, _B_PER_W)])


def _bag_sum(g, idx_rows):
    mesh = plsc.VectorSubcoreMesh(core_axis_name="c", subcore_axis_name="s")
    run = pl.kernel(
        _sc_body,
        out_type=jax.ShapeDtypeStruct((_BATCH, _OUT_PAD), jnp.float32),
        mesh=mesh,
        scratch_types=[
            pltpu.VMEM((_B_PER_W * _CHUNKS_PER_BATCH, _CHUNK), jnp.int32),
            pltpu.VMEM((_CHUNK, _OUT_PAD), jnp.float32),
            pltpu.VMEM((_B_PER_W, _OUT_PAD), jnp.float32),
            pltpu.SemaphoreType.DMA,
        ],
        compiler_params=pltpu.CompilerParams(use_tc_tiling_on_sc=False),
    )
    return run(g, idx_rows)


def kernel(inputs, table, W1, b1, W2, b2, W3, b3):
    out_dim = W3.shape[1]
    W3p = jnp.zeros((_HIDDEN, _OUT_PAD), jnp.float32).at[:, :out_dim].set(W3)
    b3p = jnp.zeros((_OUT_PAD,), jnp.float32).at[:out_dim].set(b3)

    g = _compute_g(
        table,
        W1,
        b1.reshape(1, _HIDDEN),
        W2,
        b2.reshape(1, _HIDDEN),
        W3p,
        b3p.reshape(1, _OUT_PAD),
    )

    idx_rows = inputs.astype(jnp.int32).reshape(
        _BATCH * _CHUNKS_PER_BATCH, _CHUNK
    )
    out = _bag_sum(g, idx_rows)
    return out[:, :out_dim]
